# Initial kernel scaffold; baseline (speedup 1.0000x reference)
import jax
import jax.numpy as jnp
from jax.experimental import pallas as pl


def kernel(feat_cat, feat_num, emb_table, W_num, b_num, W_dec, b_dec):
    raise NotImplementedError("write your pallas kernel here")


# speedup vs baseline: 18.4422x; 18.4422x over previous
import jax
import jax.numpy as jnp
from jax import lax
from jax.experimental import pallas as pl
from jax.experimental.pallas import tpu as pltpu
from jax.experimental.pallas import tpu_sc as plsc

B = 4096
NCAT = 26
NNUM = 10
VOCAB = 1000
CH = 128
OUT = 128
NCOLS = NCAT + NNUM

_info = plsc.get_sparse_core_info()
NC, NS, NL = _info.num_cores, _info.num_subcores, _info.num_lanes
NW = NC * NS
RW = B // NW
CB = 4
IPC = CB * NCAT
NCHUNK = RW // CB
NV = CH // NL


def _gather_sum_body(idx_hbm, table_hbm, acc_hbm, idx_v, rows_v, out_v, sem):
    wid = lax.axis_index("s") * NC + lax.axis_index("c")
    ibase = pl.multiple_of(wid * (RW * NCAT), 8)

    def chunk(ci, carry):
        off = pl.multiple_of(ibase + ci * IPC, 8)
        pltpu.sync_copy(idx_hbm.at[pl.ds(off, IPC)], idx_v)
        pltpu.async_copy(table_hbm.at[idx_v], rows_v, sem).wait()
        for r in range(CB):
            accs = [rows_v[r * NCAT, pl.ds(v * NL, NL)] for v in range(NV)]
            for c in range(1, NCAT):
                for v in range(NV):
                    accs[v] = accs[v] + rows_v[r * NCAT + c, pl.ds(v * NL, NL)]
            for v in range(NV):
                out_v[ci * CB + r, pl.ds(v * NL, NL)] = accs[v]
        return carry

    lax.fori_loop(0, NCHUNK, chunk, 0)
    obase = pl.multiple_of(wid * RW, 8)
    pltpu.sync_copy(out_v, acc_hbm.at[pl.ds(obase, RW)])


_gather_sum = pl.kernel(
    _gather_sum_body,
    out_type=jax.ShapeDtypeStruct((B, CH), jnp.float32),
    mesh=plsc.VectorSubcoreMesh(core_axis_name="c", subcore_axis_name="s"),
    scratch_types=[
        pltpu.VMEM((IPC,), jnp.int32),
        pltpu.VMEM((IPC, CH), jnp.float32),
        pltpu.VMEM((RW, CH), jnp.float32),
        pltpu.SemaphoreType.DMA,
    ],
)


def _decode_body(acc_ref, fn_ref, wn_ref, bn_ref, wd_ref, bd_ref, out_ref):
    s = acc_ref[...] + jnp.dot(fn_ref[...], wn_ref[...],
                               preferred_element_type=jnp.float32)
    s = s + jnp.sum(bn_ref[...], axis=0)[None, :]
    mean = s * (1.0 / NCOLS)
    out_ref[...] = jnp.dot(mean, wd_ref[...],
                           preferred_element_type=jnp.float32) + bd_ref[...]


_BM = 1024
_decode = pl.pallas_call(
    _decode_body,
    grid=(B // _BM,),
    in_specs=[
        pl.BlockSpec((_BM, CH), lambda i: (i, 0)),
        pl.BlockSpec((_BM, NNUM), lambda i: (i, 0)),
        pl.BlockSpec((NNUM, CH), lambda i: (0, 0)),
        pl.BlockSpec((NNUM, CH), lambda i: (0, 0)),
        pl.BlockSpec((CH, OUT), lambda i: (0, 0)),
        pl.BlockSpec((1, OUT), lambda i: (0, 0)),
    ],
    out_specs=pl.BlockSpec((_BM, OUT), lambda i: (i, 0)),
    out_shape=jax.ShapeDtypeStruct((B, OUT), jnp.float32),
)


@jax.jit
def kernel(feat_cat, feat_num, emb_table, W_num, b_num, W_dec, b_dec):
    col_off = jnp.arange(NCAT, dtype=jnp.int32) * VOCAB
    flat_idx = (feat_cat.astype(jnp.int32) + col_off[None, :]).reshape(B * NCAT)
    table = emb_table.reshape(NCAT * VOCAB, CH)
    acc = _gather_sum(flat_idx, table)
    return _decode(acc, feat_num, W_num, b_num, W_dec, b_dec.reshape(1, OUT))


# speedup vs baseline: 22.9089x; 1.2422x over previous
import jax
import jax.numpy as jnp
from jax import lax
from jax.experimental import pallas as pl
from jax.experimental.pallas import tpu as pltpu
from jax.experimental.pallas import tpu_sc as plsc

B = 4096
NCAT = 26
NNUM = 10
VOCAB = 1000
CH = 128
OUT = 128
NCOLS = NCAT + NNUM

_info = plsc.get_sparse_core_info()
NC, NS, NL = _info.num_cores, _info.num_subcores, _info.num_lanes
NW = NC * NS
RW = B // NW
CB = 4
IPC = CB * NCAT
NCHUNK = RW // CB
NV = CH // NL


NBUF = 2


def _gather_sum_body(idx_hbm, table_hbm, acc_hbm, idx_v, rows0, rows1, out_v,
                     sem0, sem1):
    wid = lax.axis_index("s") * NC + lax.axis_index("c")
    rows = (rows0, rows1)
    sems = (sem0, sem1)
    pltpu.sync_copy(idx_hbm.at[wid], idx_v)
    pltpu.async_copy(table_hbm.at[idx_v.at[0]], rows0, sem0)
    pltpu.async_copy(table_hbm.at[idx_v.at[1]], rows1, sem1)

    def outer(ci0, carry):
        for b in range(NBUF):
            ci = ci0 * NBUF + b
            pltpu.make_async_copy(table_hbm.at[idx_v.at[ci]], rows[b],
                                  sems[b]).wait()
            for r in range(CB):
                accs = [rows[b][r * NCAT, pl.ds(v * NL, NL)]
                        for v in range(NV)]
                for c in range(1, NCAT):
                    for v in range(NV):
                        accs[v] = accs[v] + rows[b][r * NCAT + c,
                                                    pl.ds(v * NL, NL)]
                for v in range(NV):
                    out_v[ci * CB + r, pl.ds(v * NL, NL)] = accs[v]
            nci = ci + NBUF

            @pl.when(nci < NCHUNK)
            def _():
                pltpu.async_copy(table_hbm.at[idx_v.at[nci]], rows[b], sems[b])
        return carry

    lax.fori_loop(0, NCHUNK // NBUF, outer, 0)
    obase = pl.multiple_of(wid * RW, 8)
    pltpu.sync_copy(out_v, acc_hbm.at[pl.ds(obase, RW)])


_gather_sum = pl.kernel(
    _gather_sum_body,
    out_type=jax.ShapeDtypeStruct((B, CH), jnp.float32),
    mesh=plsc.VectorSubcoreMesh(core_axis_name="c", subcore_axis_name="s"),
    scratch_types=[
        pltpu.VMEM((NCHUNK, IPC), jnp.int32),
        pltpu.VMEM((IPC, CH), jnp.float32),
        pltpu.VMEM((IPC, CH), jnp.float32),
        pltpu.VMEM((RW, CH), jnp.float32),
        pltpu.SemaphoreType.DMA,
        pltpu.SemaphoreType.DMA,
    ],
)


def _decode_body(acc_ref, fn_ref, wn_ref, bn_ref, wd_ref, bd_ref, out_ref):
    s = acc_ref[...] + jnp.dot(fn_ref[...], wn_ref[...],
                               preferred_element_type=jnp.float32)
    s = s + jnp.sum(bn_ref[...], axis=0)[None, :]
    mean = s * (1.0 / NCOLS)
    out_ref[...] = jnp.dot(mean, wd_ref[...],
                           preferred_element_type=jnp.float32) + bd_ref[...]


_BM = 1024
_decode = pl.pallas_call(
    _decode_body,
    grid=(B // _BM,),
    in_specs=[
        pl.BlockSpec((_BM, CH), lambda i: (i, 0)),
        pl.BlockSpec((_BM, NNUM), lambda i: (i, 0)),
        pl.BlockSpec((NNUM, CH), lambda i: (0, 0)),
        pl.BlockSpec((NNUM, CH), lambda i: (0, 0)),
        pl.BlockSpec((CH, OUT), lambda i: (0, 0)),
        pl.BlockSpec((1, OUT), lambda i: (0, 0)),
    ],
    out_specs=pl.BlockSpec((_BM, OUT), lambda i: (i, 0)),
    out_shape=jax.ShapeDtypeStruct((B, OUT), jnp.float32),
)


@jax.jit
def kernel(feat_cat, feat_num, emb_table, W_num, b_num, W_dec, b_dec):
    col_off = jnp.arange(NCAT, dtype=jnp.int32) * VOCAB
    flat_idx = (feat_cat.astype(jnp.int32) + col_off[None, :]).reshape(
        NW, NCHUNK, IPC)
    table = emb_table.reshape(NCAT * VOCAB, CH)
    acc = _gather_sum(flat_idx, table)
    return _decode(acc, feat_num, W_num, b_num, W_dec, b_dec.reshape(1, OUT))
